# baseline (speedup 1.0000x reference)
import functools

import jax
import jax.numpy as jnp
from jax import lax
from jax.experimental import pallas as pl
from jax.experimental.pallas import tpu as pltpu
from jax.experimental.pallas import tpu_sc as plsc

_NUM_CLASSES = 5
_DIM = 256
_REPEAT = 128
_LANES = 16
_NC = 2
_NS = 16
_NW = _NC * _NS
_ROWS_PER_W = _REPEAT // _NW


def _sc_body(lab_hbm, tab_hbm, out_hbm, lab_v, tab_v, buf_v):
    wid = lax.axis_index("s") * _NC + lax.axis_index("c")
    pltpu.sync_copy(lab_hbm, lab_v)
    pltpu.sync_copy(tab_hbm, tab_v)
    lv = lab_v[:]
    m = [jnp.where(lv[i] == 1, 1.0, 0.0) for i in range(_NUM_CLASSES)]
    count = m[0] + m[1] + m[2] + m[3] + m[4]
    has_active = count > 0.0
    inv = jnp.where(count > 4.5, 0.2,
          jnp.where(count > 3.5, 0.25,
          jnp.where(count > 2.5, 1.0 / 3.0,
          jnp.where(count > 1.5, 0.5, 1.0))))
    w = [inv * jnp.where(has_active, m[i], 1.0 if i == 0 else 0.0)
         for i in range(_NUM_CLASSES)]
    for c in range(_DIM // _LANES):
        sl = pl.ds(c * _LANES, _LANES)
        acc = w[0] * tab_v[0, sl]
        for i in range(1, _NUM_CLASSES):
            acc = acc + w[i] * tab_v[i, sl]
        for r in range(_ROWS_PER_W):
            buf_v[r, sl] = acc
    pltpu.sync_copy(buf_v, out_hbm.at[pl.ds(wid * _ROWS_PER_W, _ROWS_PER_W)])


@jax.jit
def _run(labels_padded, table):
    f = functools.partial(
        pl.kernel,
        mesh=plsc.VectorSubcoreMesh(core_axis_name="c", subcore_axis_name="s"),
        out_type=jax.ShapeDtypeStruct((_REPEAT, _DIM), jnp.float32),
        scratch_types=[
            pltpu.VMEM((_LANES,), jnp.int32),
            pltpu.VMEM((_NUM_CLASSES, _DIM), jnp.float32),
            pltpu.VMEM((_ROWS_PER_W, _DIM), jnp.float32),
        ],
    )(_sc_body)
    return f(labels_padded, table)


def kernel(disease_labels_batch, precomputed_embeddings):
    labels = disease_labels_batch.reshape(-1).astype(jnp.int32)
    labels_padded = jnp.pad(labels, (0, _LANES - _NUM_CLASSES))
    out = _run(labels_padded, precomputed_embeddings)
    return out[None, :, :]


# speedup vs baseline: 1.1123x; 1.1123x over previous
import functools

import jax
import jax.numpy as jnp
from jax import lax
from jax.experimental import pallas as pl
from jax.experimental.pallas import tpu as pltpu
from jax.experimental.pallas import tpu_sc as plsc

_NUM_CLASSES = 5
_DIM = 256
_REPEAT = 128
_LANES = 16
_NW = 16
_ROWS_PER_W = _REPEAT // _NW


def _sc_body(lab_hbm, tab_hbm, out_hbm, lab_v, tab_v, buf_v):
    wid = lax.axis_index("s")
    pltpu.sync_copy(lab_hbm, lab_v.at[pl.ds(0, _NUM_CLASSES)])
    pltpu.sync_copy(tab_hbm, tab_v)
    lv = lab_v[:]
    m = [jnp.where(lv[i] == 1, 1.0, 0.0) for i in range(_NUM_CLASSES)]
    count = m[0] + m[1] + m[2] + m[3] + m[4]
    has_active = count > 0.0
    inv = jnp.where(count > 4.5, 0.2,
          jnp.where(count > 3.5, 0.25,
          jnp.where(count > 2.5, 1.0 / 3.0,
          jnp.where(count > 1.5, 0.5, 1.0))))
    w = [inv * jnp.where(has_active, m[i], 1.0 if i == 0 else 0.0)
         for i in range(_NUM_CLASSES)]
    for c in range(_DIM // _LANES):
        sl = pl.ds(c * _LANES, _LANES)
        acc = w[0] * tab_v[0, sl]
        for i in range(1, _NUM_CLASSES):
            acc = acc + w[i] * tab_v[i, sl]
        for r in range(_ROWS_PER_W):
            buf_v[r, sl] = acc
    pltpu.sync_copy(buf_v,
                    out_hbm.at[0, pl.ds(wid * _ROWS_PER_W, _ROWS_PER_W)])


@jax.jit
def _run(labels, table):
    f = functools.partial(
        pl.kernel,
        mesh=plsc.VectorSubcoreMesh(core_axis_name="c", subcore_axis_name="s",
                                    num_cores=1),
        out_type=jax.ShapeDtypeStruct((1, _REPEAT, _DIM), jnp.float32),
        scratch_types=[
            pltpu.VMEM((_LANES,), jnp.int32),
            pltpu.VMEM((_NUM_CLASSES, _DIM), jnp.float32),
            pltpu.VMEM((_ROWS_PER_W, _DIM), jnp.float32),
        ],
    )(_sc_body)
    return f(labels, table)


def kernel(disease_labels_batch, precomputed_embeddings):
    labels = disease_labels_batch.reshape(-1).astype(jnp.int32)
    return _run(labels, precomputed_embeddings)
